# baseline (speedup 1.0000x reference)
import dataclasses
import functools

import jax
import jax.numpy as jnp
from jax import lax
from jax.experimental import pallas as pl
from jax.experimental.pallas import tpu as pltpu
from jax.experimental.pallas import tpu_sc as plsc

HIDDEN = 2048
NUM_EXPERTS = 8
FF = 768
TOP_K = 2
T = 4096
SLOTS = T * TOP_K
BM = 256
BMS = 512
BMS_SHIFT = 9
MAXB = SLOTS // BMS + NUM_EXPERTS - 1
SPAD = MAXB * BMS
NBE = 48
NC = 2
NS = 16
L = 16
CHUNK = SLOTS // (NC * NS)


def _sc_compiler_params():
    cp = pltpu.CompilerParams()
    if "needs_layout_passes" in pltpu.CompilerParams.__dataclass_fields__:
        cp = dataclasses.replace(cp, needs_layout_passes=False)
    return cp


def _router_body(x_ref, rw_ref, logits_ref, p_ref, i_ref, h_ref):
    x = x_ref[...]
    rw = rw_ref[...]
    logits = lax.dot_general(
        x, rw, (((1,), (1,)), ((), ())), preferred_element_type=jnp.float32)
    logits_ref[...] = logits
    m = jnp.max(logits, axis=1, keepdims=True)
    ex = jnp.exp(logits - m)
    probs = ex / jnp.sum(ex, axis=1, keepdims=True)
    iota = lax.broadcasted_iota(jnp.int32, probs.shape, 1)
    m1 = jnp.max(probs, axis=1, keepdims=True)
    idx1 = jnp.min(jnp.where(probs == m1, iota, NUM_EXPERTS), axis=1,
                   keepdims=True)
    probs_m = jnp.where(iota == idx1, -1.0, probs)
    m2 = jnp.max(probs_m, axis=1, keepdims=True)
    idx2 = jnp.min(jnp.where(probs_m == m2, iota, NUM_EXPERTS), axis=1,
                   keepdims=True)
    s = m1 + m2
    p_ref[...] = jnp.concatenate([m1 / s, m2 / s], axis=1)
    i_ref[...] = jnp.concatenate([idx1, idx2], axis=1)
    pad = jnp.zeros((1, 2 * NUM_EXPERTS - NUM_EXPERTS), jnp.int32)
    h0 = jnp.concatenate(
        [jnp.sum((iota == idx1).astype(jnp.int32), axis=0, keepdims=True),
         pad], axis=1)
    h1 = jnp.concatenate(
        [jnp.sum((iota == idx2).astype(jnp.int32), axis=0, keepdims=True),
         pad], axis=1)
    h_ref[0] = jnp.concatenate([h0, h1], axis=0)


def _router(flat, router_weight):
    tb = T // BM
    return pl.pallas_call(
        _router_body,
        grid=(tb,),
        in_specs=[
            pl.BlockSpec((BM, HIDDEN), lambda t: (t, 0)),
            pl.BlockSpec((NUM_EXPERTS, HIDDEN), lambda t: (0, 0)),
        ],
        out_specs=[
            pl.BlockSpec((BM, NUM_EXPERTS), lambda t: (t, 0)),
            pl.BlockSpec((BM, TOP_K), lambda t: (t, 0)),
            pl.BlockSpec((BM, TOP_K), lambda t: (t, 0)),
            pl.BlockSpec((1, 2, 2 * NUM_EXPERTS), lambda t: (t, 0, 0)),
        ],
        out_shape=[
            jax.ShapeDtypeStruct((T, NUM_EXPERTS), jnp.float32),
            jax.ShapeDtypeStruct((T, TOP_K), jnp.float32),
            jax.ShapeDtypeStruct((T, TOP_K), jnp.int32),
            jax.ShapeDtypeStruct((T // BM, 2, 2 * NUM_EXPERTS), jnp.int32),
        ],
    )(flat, router_weight)


def _permute_body(ids_hbm, hist_hbm, x_hbm, sx_hbm, dest_hbm, be_hbm,
                  ids_v, hist_v, dest_v, bev_v, buf0, buf1, buf2,
                  sem_in, gsem0, gsem1, gsem2, ssem0, ssem1, ssem2):
    c = lax.axis_index("c")
    s = lax.axis_index("s")
    my_chunk = 2 * s + c
    base_slot = my_chunk * CHUNK
    iota16 = lax.iota(jnp.int32, L)
    zeros16 = jnp.zeros((L,), jnp.int32)

    pltpu.async_copy(hist_hbm, hist_v, sem_in).wait()
    pltpu.async_copy(ids_hbm.at[pl.ds(base_slot, CHUNK)], ids_v,
                     sem_in).wait()

    total = zeros16
    before = zeros16
    for j in range(NC * NS):
        jj = 2 * j if j < NS else 2 * (j - NS) + 1
        row = hist_v[jj, :]
        total = total + row
        before = before + jnp.where(my_chunk > j, row, zeros16)

    nblocks = (total + (BMS - 1)) >> BMS_SHIFT
    bstart = plsc.cumsum(nblocks) - nblocks
    mybase = bstart * BMS + before

    for j in range(NBE // L):
        bvec = iota16 + (j * L)
        cnt = zeros16
        for e in range(NUM_EXPERTS):
            se = jnp.sum(jnp.where(iota16 == e, bstart, zeros16))
            cnt = cnt + jnp.where(bvec >= se, 1, 0)
        bev_v[pl.ds(j * L, L)] = cnt - 1

    @pl.when(jnp.logical_and(c == 0, s == 0))
    def _():
        pltpu.sync_copy(bev_v, be_hbm)

    for i in range(CHUNK // L):
        v = ids_v[pl.ds(i * L, L)]
        dest = zeros16
        for e in range(NUM_EXPERTS):
            mask = v == e
            mi = jnp.where(mask, 1, 0)
            rank = plsc.cumsum(mi) - 1
            base_e = jnp.sum(jnp.where(iota16 == e, mybase, zeros16))
            dest = jnp.where(mask, base_e + rank, dest)
            mybase = mybase + jnp.where(iota16 == e, jnp.sum(mi), zeros16)
        dest_v[pl.ds(i * L, L)] = dest

    pltpu.sync_copy(dest_v, dest_hbm.at[pl.ds(base_slot, CHUNK)])

    bufs = [buf0, buf1, buf2]
    gsems = [gsem0, gsem1, gsem2]
    ssems = [ssem0, ssem1, ssem2]
    nd = 3
    niter = CHUNK // L
    gh = [None] * nd
    sh = [None] * nd

    def tokv(i):
        return (iota16 + (base_slot + i * L)) & (T - 1)

    for i in range(nd - 1):
        gh[i] = pltpu.async_copy(x_hbm.at[tokv(i)], bufs[i], gsems[i])
    for i in range(niter):
        b = i % nd
        gh[b].wait()
        dv = dest_v[pl.ds(i * L, L)]
        sh[b] = pltpu.async_copy(bufs[b], sx_hbm.at[dv], ssems[b])
        if i + nd - 1 < niter:
            nb = (i + nd - 1) % nd
            if sh[nb] is not None:
                sh[nb].wait()
            gh[nb] = pltpu.async_copy(x_hbm.at[tokv(i + nd - 1)], bufs[nb],
                                      gsems[nb])
    for b in range(nd):
        if sh[b] is not None:
            sh[b].wait()


def _permute(ids_flat, hist, x_f32):
    mesh = plsc.VectorSubcoreMesh(core_axis_name="c", subcore_axis_name="s",
                                  num_cores=NC, num_subcores=NS)
    k = pl.kernel(
        _permute_body,
        out_type=[
            jax.ShapeDtypeStruct((SPAD, HIDDEN), jnp.float32),
            jax.ShapeDtypeStruct((SLOTS,), jnp.int32),
            jax.ShapeDtypeStruct((NBE,), jnp.int32),
        ],
        mesh=mesh,
        scratch_types=[
            pltpu.VMEM((CHUNK,), jnp.int32),
            pltpu.VMEM((NC * NS, L), jnp.int32),
            pltpu.VMEM((CHUNK,), jnp.int32),
            pltpu.VMEM((NBE,), jnp.int32),
            pltpu.VMEM((L, HIDDEN), jnp.float32),
            pltpu.VMEM((L, HIDDEN), jnp.float32),
            pltpu.VMEM((L, HIDDEN), jnp.float32),
            pltpu.SemaphoreType.DMA,
            pltpu.SemaphoreType.DMA,
            pltpu.SemaphoreType.DMA,
            pltpu.SemaphoreType.DMA,
            pltpu.SemaphoreType.DMA,
            pltpu.SemaphoreType.DMA,
            pltpu.SemaphoreType.DMA,
        ],
        compiler_params=_sc_compiler_params(),
    )
    return k(ids_flat, hist, x_f32)


def _gemm_body(be_ref, x_ref, w1_ref, w2_ref, y_ref):
    del be_ref
    h = lax.dot_general(
        x_ref[...], w1_ref[0], (((1,), (1,)), ((), ())),
        preferred_element_type=jnp.float32)
    gate = h[:, :FF]
    up = h[:, FF:]
    act = gate * jax.nn.sigmoid(gate) * up
    y = lax.dot_general(
        act, w2_ref[0], (((1,), (1,)), ((), ())),
        preferred_element_type=jnp.float32)
    y_ref[...] = y


def _gemm(block_expert, sorted_x, w1b, w2b):
    grid_spec = pltpu.PrefetchScalarGridSpec(
        num_scalar_prefetch=1,
        grid=(MAXB,),
        in_specs=[
            pl.BlockSpec((BMS, HIDDEN), lambda b, be: (b, 0)),
            pl.BlockSpec((1, 2 * FF, HIDDEN), lambda b, be: (be[b], 0, 0)),
            pl.BlockSpec((1, HIDDEN, FF), lambda b, be: (be[b], 0, 0)),
        ],
        out_specs=pl.BlockSpec((BMS, HIDDEN), lambda b, be: (b, 0)),
    )
    return pl.pallas_call(
        _gemm_body,
        grid_spec=grid_spec,
        out_shape=jax.ShapeDtypeStruct((SPAD, HIDDEN), jnp.float32),
    )(block_expert, sorted_x, w1b, w2b)


def _unpermute_body(dest_hbm, y_hbm, out_hbm,
                    dest_v, buf0, buf1, buf2,
                    sem_in, gsem0, gsem1, gsem2, ssem0, ssem1, ssem2):
    c = lax.axis_index("c")
    s = lax.axis_index("s")
    base_slot = (2 * s + c) * CHUNK
    pltpu.async_copy(dest_hbm.at[pl.ds(base_slot, CHUNK)], dest_v,
                     sem_in).wait()
    bufs = [buf0, buf1, buf2]
    gsems = [gsem0, gsem1, gsem2]
    ssems = [ssem0, ssem1, ssem2]
    nd = 3
    niter = CHUNK // L
    gh = [None] * nd
    sh = [None] * nd
    for i in range(nd - 1):
        gh[i] = pltpu.async_copy(y_hbm.at[dest_v.at[pl.ds(i * L, L)]],
                                 bufs[i], gsems[i])
    for i in range(niter):
        b = i % nd
        gh[b].wait()
        sh[b] = pltpu.async_copy(
            bufs[b], out_hbm.at[pl.ds(base_slot + i * L, L)], ssems[b])
        if i + nd - 1 < niter:
            nb = (i + nd - 1) % nd
            if sh[nb] is not None:
                sh[nb].wait()
            gh[nb] = pltpu.async_copy(
                y_hbm.at[dest_v.at[pl.ds((i + nd - 1) * L, L)]], bufs[nb],
                gsems[nb])
    for b in range(nd):
        if sh[b] is not None:
            sh[b].wait()


def _unpermute(dest, y_sorted):
    mesh = plsc.VectorSubcoreMesh(core_axis_name="c", subcore_axis_name="s",
                                  num_cores=NC, num_subcores=NS)
    k = pl.kernel(
        _unpermute_body,
        out_type=jax.ShapeDtypeStruct((SLOTS, HIDDEN), jnp.float32),
        mesh=mesh,
        scratch_types=[
            pltpu.VMEM((CHUNK,), jnp.int32),
            pltpu.VMEM((L, HIDDEN), jnp.float32),
            pltpu.VMEM((L, HIDDEN), jnp.float32),
            pltpu.VMEM((L, HIDDEN), jnp.float32),
            pltpu.SemaphoreType.DMA,
            pltpu.SemaphoreType.DMA,
            pltpu.SemaphoreType.DMA,
            pltpu.SemaphoreType.DMA,
            pltpu.SemaphoreType.DMA,
            pltpu.SemaphoreType.DMA,
            pltpu.SemaphoreType.DMA,
        ],
        compiler_params=_sc_compiler_params(),
    )
    return k(dest, y_sorted)


def _combine_body(y0_ref, y1_ref, p_ref, out_ref):
    p = p_ref[...]
    out_ref[...] = p[:, 0:1] * y0_ref[...] + p[:, 1:2] * y1_ref[...]


def _combine(y_slot, p):
    tb = T // BM
    return pl.pallas_call(
        _combine_body,
        grid=(tb,),
        in_specs=[
            pl.BlockSpec((BM, HIDDEN), lambda t: (t, 0)),
            pl.BlockSpec((BM, HIDDEN), lambda t: (t + T // BM, 0)),
            pl.BlockSpec((BM, TOP_K), lambda t: (t, 0)),
        ],
        out_specs=pl.BlockSpec((BM, HIDDEN), lambda t: (t, 0)),
        out_shape=jax.ShapeDtypeStruct((T, HIDDEN), jnp.float32),
    )(y_slot, y_slot, p)


@jax.jit
def kernel(hidden_states, router_weight, w1, w2):
    b, s, d = hidden_states.shape
    flat = hidden_states.reshape(-1, d)
    logits, p, idx, hist = _router(flat, router_weight)
    ids_flat = jnp.concatenate([idx[:, 0], idx[:, 1]])
    hist2d = hist.reshape(NC * NS, 2 * NUM_EXPERTS)
    sorted_x, dest, block_expert = _permute(ids_flat, hist2d, flat)
    y_sorted = _gemm(block_expert, sorted_x, w1, w2)
    y_slot = _unpermute(dest, y_sorted)
    out = _combine(y_slot, p)
    return out.reshape(b, s, d), logits


# speedup vs baseline: 1.0238x; 1.0238x over previous
import dataclasses
import functools

import jax
import jax.numpy as jnp
from jax import lax
from jax.experimental import pallas as pl
from jax.experimental.pallas import tpu as pltpu
from jax.experimental.pallas import tpu_sc as plsc

HIDDEN = 2048
NUM_EXPERTS = 8
FF = 768
TOP_K = 2
T = 4096
SLOTS = T * TOP_K
BM = 256
BMS = 256
BMS_SHIFT = 8
MAXB = SLOTS // BMS + NUM_EXPERTS - 1
SPAD = MAXB * BMS
NBE = 48
NC = 2
NS = 16
L = 16
CHUNK = SLOTS // (NC * NS)


def _sc_compiler_params():
    cp = pltpu.CompilerParams()
    if "needs_layout_passes" in pltpu.CompilerParams.__dataclass_fields__:
        cp = dataclasses.replace(cp, needs_layout_passes=False)
    return cp


def _router_body(x_ref, rw_ref, logits_ref, p_ref, i_ref, h_ref):
    x = x_ref[...]
    rw = rw_ref[...]
    logits = lax.dot_general(
        x, rw, (((1,), (1,)), ((), ())), preferred_element_type=jnp.float32)
    logits_ref[...] = logits
    m = jnp.max(logits, axis=1, keepdims=True)
    ex = jnp.exp(logits - m)
    probs = ex / jnp.sum(ex, axis=1, keepdims=True)
    iota = lax.broadcasted_iota(jnp.int32, probs.shape, 1)
    m1 = jnp.max(probs, axis=1, keepdims=True)
    idx1 = jnp.min(jnp.where(probs == m1, iota, NUM_EXPERTS), axis=1,
                   keepdims=True)
    probs_m = jnp.where(iota == idx1, -1.0, probs)
    m2 = jnp.max(probs_m, axis=1, keepdims=True)
    idx2 = jnp.min(jnp.where(probs_m == m2, iota, NUM_EXPERTS), axis=1,
                   keepdims=True)
    s = m1 + m2
    p_ref[...] = jnp.concatenate([m1 / s, m2 / s], axis=1)
    i_ref[...] = jnp.concatenate([idx1, idx2], axis=1)
    pad = jnp.zeros((1, 2 * NUM_EXPERTS - NUM_EXPERTS), jnp.int32)
    h0 = jnp.concatenate(
        [jnp.sum((iota == idx1).astype(jnp.int32), axis=0, keepdims=True),
         pad], axis=1)
    h1 = jnp.concatenate(
        [jnp.sum((iota == idx2).astype(jnp.int32), axis=0, keepdims=True),
         pad], axis=1)
    h_ref[0] = jnp.concatenate([h0, h1], axis=0)


def _router(flat, router_weight):
    tb = T // BM
    return pl.pallas_call(
        _router_body,
        grid=(tb,),
        in_specs=[
            pl.BlockSpec((BM, HIDDEN), lambda t: (t, 0)),
            pl.BlockSpec((NUM_EXPERTS, HIDDEN), lambda t: (0, 0)),
        ],
        out_specs=[
            pl.BlockSpec((BM, NUM_EXPERTS), lambda t: (t, 0)),
            pl.BlockSpec((BM, TOP_K), lambda t: (t, 0)),
            pl.BlockSpec((BM, TOP_K), lambda t: (t, 0)),
            pl.BlockSpec((1, 2, 2 * NUM_EXPERTS), lambda t: (t, 0, 0)),
        ],
        out_shape=[
            jax.ShapeDtypeStruct((T, NUM_EXPERTS), jnp.float32),
            jax.ShapeDtypeStruct((T, TOP_K), jnp.float32),
            jax.ShapeDtypeStruct((T, TOP_K), jnp.int32),
            jax.ShapeDtypeStruct((T // BM, 2, 2 * NUM_EXPERTS), jnp.int32),
        ],
    )(flat, router_weight)


def _permute_body(ids_hbm, hist_hbm, x_hbm, sx_hbm, dest_hbm, be_hbm,
                  ids_v, hist_v, dest_v, bev_v, buf0, buf1, buf2,
                  sem_in, gsem0, gsem1, gsem2, ssem0, ssem1, ssem2):
    c = lax.axis_index("c")
    s = lax.axis_index("s")
    my_chunk = 2 * s + c
    base_slot = my_chunk * CHUNK
    iota16 = lax.iota(jnp.int32, L)
    zeros16 = jnp.zeros((L,), jnp.int32)

    pltpu.async_copy(hist_hbm, hist_v, sem_in).wait()
    pltpu.async_copy(ids_hbm.at[pl.ds(base_slot, CHUNK)], ids_v,
                     sem_in).wait()

    total = zeros16
    before = zeros16
    for j in range(NC * NS):
        jj = 2 * j if j < NS else 2 * (j - NS) + 1
        row = hist_v[jj, :]
        total = total + row
        before = before + jnp.where(my_chunk > j, row, zeros16)

    nblocks = (total + (BMS - 1)) >> BMS_SHIFT
    bstart = plsc.cumsum(nblocks) - nblocks
    mybase = bstart * BMS + before

    for j in range(NBE // L):
        bvec = iota16 + (j * L)
        cnt = zeros16
        for e in range(NUM_EXPERTS):
            se = jnp.sum(jnp.where(iota16 == e, bstart, zeros16))
            cnt = cnt + jnp.where(bvec >= se, 1, 0)
        bev_v[pl.ds(j * L, L)] = cnt - 1

    @pl.when(jnp.logical_and(c == 0, s == 0))
    def _():
        pltpu.sync_copy(bev_v, be_hbm)

    for i in range(CHUNK // L):
        v = ids_v[pl.ds(i * L, L)]
        dest = zeros16
        for e in range(NUM_EXPERTS):
            mask = v == e
            mi = jnp.where(mask, 1, 0)
            rank = plsc.cumsum(mi) - 1
            base_e = jnp.sum(jnp.where(iota16 == e, mybase, zeros16))
            dest = jnp.where(mask, base_e + rank, dest)
            mybase = mybase + jnp.where(iota16 == e, jnp.sum(mi), zeros16)
        dest_v[pl.ds(i * L, L)] = dest

    pltpu.sync_copy(dest_v, dest_hbm.at[pl.ds(base_slot, CHUNK)])

    bufs = [buf0, buf1, buf2]
    gsems = [gsem0, gsem1, gsem2]
    ssems = [ssem0, ssem1, ssem2]
    nd = 3
    niter = CHUNK // L
    gh = [None] * nd
    sh = [None] * nd

    def tokv(i):
        return (iota16 + (base_slot + i * L)) & (T - 1)

    for i in range(nd - 1):
        gh[i] = pltpu.async_copy(x_hbm.at[tokv(i)], bufs[i], gsems[i])
    for i in range(niter):
        b = i % nd
        gh[b].wait()
        dv = dest_v[pl.ds(i * L, L)]
        sh[b] = pltpu.async_copy(bufs[b], sx_hbm.at[dv], ssems[b])
        if i + nd - 1 < niter:
            nb = (i + nd - 1) % nd
            if sh[nb] is not None:
                sh[nb].wait()
            gh[nb] = pltpu.async_copy(x_hbm.at[tokv(i + nd - 1)], bufs[nb],
                                      gsems[nb])
    for b in range(nd):
        if sh[b] is not None:
            sh[b].wait()


def _permute(ids_flat, hist, x_f32):
    mesh = plsc.VectorSubcoreMesh(core_axis_name="c", subcore_axis_name="s",
                                  num_cores=NC, num_subcores=NS)
    k = pl.kernel(
        _permute_body,
        out_type=[
            jax.ShapeDtypeStruct((SPAD, HIDDEN), jnp.float32),
            jax.ShapeDtypeStruct((SLOTS,), jnp.int32),
            jax.ShapeDtypeStruct((NBE,), jnp.int32),
        ],
        mesh=mesh,
        scratch_types=[
            pltpu.VMEM((CHUNK,), jnp.int32),
            pltpu.VMEM((NC * NS, L), jnp.int32),
            pltpu.VMEM((CHUNK,), jnp.int32),
            pltpu.VMEM((NBE,), jnp.int32),
            pltpu.VMEM((L, HIDDEN), jnp.float32),
            pltpu.VMEM((L, HIDDEN), jnp.float32),
            pltpu.VMEM((L, HIDDEN), jnp.float32),
            pltpu.SemaphoreType.DMA,
            pltpu.SemaphoreType.DMA,
            pltpu.SemaphoreType.DMA,
            pltpu.SemaphoreType.DMA,
            pltpu.SemaphoreType.DMA,
            pltpu.SemaphoreType.DMA,
            pltpu.SemaphoreType.DMA,
        ],
        compiler_params=_sc_compiler_params(),
    )
    return k(ids_flat, hist, x_f32)


def _gemm_body(be_ref, x_ref, w1_ref, w2_ref, y_ref):
    del be_ref
    h = lax.dot_general(
        x_ref[...], w1_ref[0], (((1,), (1,)), ((), ())),
        preferred_element_type=jnp.float32)
    gate = h[:, :FF]
    up = h[:, FF:]
    act = gate * jax.nn.sigmoid(gate) * up
    y = lax.dot_general(
        act, w2_ref[0], (((1,), (1,)), ((), ())),
        preferred_element_type=jnp.float32)
    y_ref[...] = y


def _gemm(block_expert, sorted_x, w1b, w2b):
    grid_spec = pltpu.PrefetchScalarGridSpec(
        num_scalar_prefetch=1,
        grid=(MAXB,),
        in_specs=[
            pl.BlockSpec((BMS, HIDDEN), lambda b, be: (b, 0)),
            pl.BlockSpec((1, 2 * FF, HIDDEN), lambda b, be: (be[b], 0, 0)),
            pl.BlockSpec((1, HIDDEN, FF), lambda b, be: (be[b], 0, 0)),
        ],
        out_specs=pl.BlockSpec((BMS, HIDDEN), lambda b, be: (b, 0)),
    )
    return pl.pallas_call(
        _gemm_body,
        grid_spec=grid_spec,
        out_shape=jax.ShapeDtypeStruct((SPAD, HIDDEN), jnp.float32),
    )(block_expert, sorted_x, w1b, w2b)


def _unpermute_body(dest_hbm, y_hbm, out_hbm,
                    dest_v, buf0, buf1, buf2,
                    sem_in, gsem0, gsem1, gsem2, ssem0, ssem1, ssem2):
    c = lax.axis_index("c")
    s = lax.axis_index("s")
    base_slot = (2 * s + c) * CHUNK
    pltpu.async_copy(dest_hbm.at[pl.ds(base_slot, CHUNK)], dest_v,
                     sem_in).wait()
    bufs = [buf0, buf1, buf2]
    gsems = [gsem0, gsem1, gsem2]
    ssems = [ssem0, ssem1, ssem2]
    nd = 3
    niter = CHUNK // L
    gh = [None] * nd
    sh = [None] * nd
    for i in range(nd - 1):
        gh[i] = pltpu.async_copy(y_hbm.at[dest_v.at[pl.ds(i * L, L)]],
                                 bufs[i], gsems[i])
    for i in range(niter):
        b = i % nd
        gh[b].wait()
        sh[b] = pltpu.async_copy(
            bufs[b], out_hbm.at[pl.ds(base_slot + i * L, L)], ssems[b])
        if i + nd - 1 < niter:
            nb = (i + nd - 1) % nd
            if sh[nb] is not None:
                sh[nb].wait()
            gh[nb] = pltpu.async_copy(
                y_hbm.at[dest_v.at[pl.ds((i + nd - 1) * L, L)]], bufs[nb],
                gsems[nb])
    for b in range(nd):
        if sh[b] is not None:
            sh[b].wait()


def _unpermute(dest, y_sorted):
    mesh = plsc.VectorSubcoreMesh(core_axis_name="c", subcore_axis_name="s",
                                  num_cores=NC, num_subcores=NS)
    k = pl.kernel(
        _unpermute_body,
        out_type=jax.ShapeDtypeStruct((SLOTS, HIDDEN), jnp.float32),
        mesh=mesh,
        scratch_types=[
            pltpu.VMEM((CHUNK,), jnp.int32),
            pltpu.VMEM((L, HIDDEN), jnp.float32),
            pltpu.VMEM((L, HIDDEN), jnp.float32),
            pltpu.VMEM((L, HIDDEN), jnp.float32),
            pltpu.SemaphoreType.DMA,
            pltpu.SemaphoreType.DMA,
            pltpu.SemaphoreType.DMA,
            pltpu.SemaphoreType.DMA,
            pltpu.SemaphoreType.DMA,
            pltpu.SemaphoreType.DMA,
            pltpu.SemaphoreType.DMA,
        ],
        compiler_params=_sc_compiler_params(),
    )
    return k(dest, y_sorted)


def _combine_body(y0_ref, y1_ref, p_ref, out_ref):
    p = p_ref[...]
    out_ref[...] = p[:, 0:1] * y0_ref[...] + p[:, 1:2] * y1_ref[...]


def _combine(y_slot, p):
    tb = T // BM
    return pl.pallas_call(
        _combine_body,
        grid=(tb,),
        in_specs=[
            pl.BlockSpec((BM, HIDDEN), lambda t: (t, 0)),
            pl.BlockSpec((BM, HIDDEN), lambda t: (t + T // BM, 0)),
            pl.BlockSpec((BM, TOP_K), lambda t: (t, 0)),
        ],
        out_specs=pl.BlockSpec((BM, HIDDEN), lambda t: (t, 0)),
        out_shape=jax.ShapeDtypeStruct((T, HIDDEN), jnp.float32),
    )(y_slot, y_slot, p)


@jax.jit
def kernel(hidden_states, router_weight, w1, w2):
    b, s, d = hidden_states.shape
    flat = hidden_states.reshape(-1, d)
    logits, p, idx, hist = _router(flat, router_weight)
    ids_flat = jnp.concatenate([idx[:, 0], idx[:, 1]])
    hist2d = hist.reshape(NC * NS, 2 * NUM_EXPERTS)
    sorted_x, dest, block_expert = _permute(ids_flat, hist2d, flat)
    y_sorted = _gemm(block_expert, sorted_x, w1, w2)
    y_slot = _unpermute(dest, y_sorted)
    out = _combine(y_slot, p)
    return out.reshape(b, s, d), logits
